# baseline (speedup 1.0000x reference)
import jax
import jax.numpy as jnp
from jax import lax
from jax.experimental import pallas as pl
from jax.experimental.pallas import tpu as pltpu
from jax.experimental.pallas import tpu_sc as plsc

N_NODES = 10000
N_EDGES = 320000
N_GRAPHS = 64
D = 128
LAT = 64

NC = 2
NS = 16
NW = NC * NS

EW = N_EDGES // NW
EK = 80
ECH = 128
EWP = ECH * EK
IBCH = 8
NBLK = ECH // IBCH
NB = 3
ADUMP = 8

PK = 104
PCH = 3
PW = PK * PCH
PTAIL = N_NODES - PW * NW

_SC_MESH = plsc.VectorSubcoreMesh(core_axis_name="c", subcore_axis_name="s")


ZR = 624
ZTAIL = N_NODES - NS * ZR


def _sc_agg_body(h_hbm, src_hbm, dst_hbm, zeros_hbm, out_hbm,
                 src_v, dst_v, rows_v, acc,
                 sem_ga, sem_gb, sem_gc, sem_sa, sem_sb, sem_sc,
                 sem_ia, sem_ib):
    c = lax.axis_index("c")
    s = lax.axis_index("s")

    @pl.when(c == 0)
    def _():
        pltpu.sync_copy(h_hbm.at[pl.ds(s * ZR, ZR)],
                        acc.at[pl.ds(s * ZR, ZR)])

        @pl.when(s == 0)
        def _():
            pltpu.sync_copy(h_hbm.at[pl.ds(NS * ZR, ZTAIL)],
                            acc.at[pl.ds(NS * ZR, ZTAIL)])

    @pl.when(c == 1)
    def _():
        pltpu.sync_copy(zeros_hbm.at[pl.ds(s * ZR, ZR)],
                        acc.at[pl.ds(s * ZR, ZR)])

        @pl.when(s == 0)
        def _():
            pltpu.sync_copy(zeros_hbm.at[pl.ds(NS * ZR, ZTAIL)],
                            acc.at[pl.ds(NS * ZR, ZTAIL)])

    rows = tuple(rows_v.at[i] for i in range(NB))
    gsem = (sem_ga, sem_gb, sem_gc)
    ssem = (sem_sa, sem_sb, sem_sc)
    srcb = (src_v.at[0], src_v.at[1])
    dstb = (dst_v.at[0], dst_v.at[1])

    def _idx_load(blk, sync=False):
        p = blk % 2
        sl = pl.ds(blk * IBCH, IBCH)
        if sync:
            pltpu.sync_copy(src_hbm.at[c, s, sl], srcb[p])
            pltpu.sync_copy(dst_hbm.at[c, s, sl], dstb[p])
        else:
            pltpu.async_copy(src_hbm.at[c, s, sl], srcb[p], sem_ia)
            pltpu.async_copy(dst_hbm.at[c, s, sl], dstb[p], sem_ib)

    def _idx_wait():
        sl = pl.ds(0, IBCH)
        pltpu.make_async_copy(src_hbm.at[c, s, sl], srcb[0], sem_ia).wait()
        pltpu.make_async_copy(dst_hbm.at[c, s, sl], dstb[0], sem_ib).wait()

    def _gather(k):
        p, j, b = (k // IBCH) % 2, k % IBCH, k % NB
        pltpu.async_copy(h_hbm.at[srcb[p].at[j]], rows[b], gsem[b])

    def _scatter(k):
        p, j, b = (k // IBCH) % 2, k % IBCH, k % NB
        pltpu.async_copy(rows[b], acc.at[dstb[p].at[j]], ssem[b], add=True)

    def _wait_g(k):
        b = k % NB
        pltpu.make_async_copy(h_hbm.at[srcb[0].at[0]], rows[b], gsem[b]).wait()

    def _wait_s(k):
        b = k % NB
        pltpu.make_async_copy(rows[b], acc.at[dstb[0].at[0]], ssem[b]).wait()

    _idx_load(0, sync=True)
    _idx_load(1)
    plsc.subcore_barrier()

    _gather(0)
    _gather(1)
    for k in range(ECH):
        _wait_g(k)
        _scatter(k)
        if k + 2 < ECH:
            if (k + 2) % IBCH == 0:
                _idx_wait()
            if (k + 1) % IBCH == 0:
                nblk = (k + 1) // IBCH + 1
                if nblk < NBLK:
                    _idx_load(nblk)
            if k >= 1:
                _wait_s(k - 1)
            _gather(k + 2)
    _wait_s(ECH - 2)
    _wait_s(ECH - 1)
    plsc.subcore_barrier()
    pltpu.sync_copy(acc.at[pl.ds(s * ZR, ZR)],
                    out_hbm.at[c, pl.ds(s * ZR, ZR)])

    @pl.when(s == 0)
    def _():
        pltpu.sync_copy(acc.at[pl.ds(NS * ZR, ZTAIL)],
                        out_hbm.at[c, pl.ds(NS * ZR, ZTAIL)])


_sc_agg = pl.kernel(
    _sc_agg_body,
    out_type=jax.ShapeDtypeStruct((NC, N_NODES, D), jnp.float32),
    mesh=_SC_MESH,
    scratch_types=[
        pltpu.VMEM((2, IBCH, EK), jnp.int32),
        pltpu.VMEM((2, IBCH, EK), jnp.int32),
        pltpu.VMEM((NB, EK, D), jnp.float32),
        pltpu.VMEM_SHARED((N_NODES + ADUMP, D), jnp.float32),
        pltpu.SemaphoreType.DMA,
        pltpu.SemaphoreType.DMA,
        pltpu.SemaphoreType.DMA,
        pltpu.SemaphoreType.DMA,
        pltpu.SemaphoreType.DMA,
        pltpu.SemaphoreType.DMA,
        pltpu.SemaphoreType.DMA,
        pltpu.SemaphoreType.DMA,
    ],
)


_MLP_R = 1000


def _mlp_body(aa_ref, ab_ref, w1_ref, b1_ref, gs_ref, be_ref,
              w2_ref, b2_ref, out_ref):
    h = aa_ref[...] + ab_ref[...]
    t = jnp.dot(h, w1_ref[...], preferred_element_type=jnp.float32)
    t = t + b1_ref[...]
    t = jnp.where(t >= 0, t, 0.2 * t)
    t = t * gs_ref[...] + be_ref[...]
    t = jnp.dot(t, w2_ref[...], preferred_element_type=jnp.float32)
    t = t + b2_ref[...]
    out_ref[...] = jnp.where(t >= 0, t, 0.2 * t)


def _tc_mlp(agg_a, agg_b, w1, b1, gs, be, w2, b2):
    grid = (N_NODES // _MLP_R,)
    row_spec = pl.BlockSpec((_MLP_R, D), lambda i: (i, 0))
    mat_spec = pl.BlockSpec((D, D), lambda i: (0, 0))
    vec_spec = pl.BlockSpec((1, D), lambda i: (0, 0))
    return pl.pallas_call(
        _mlp_body,
        grid=grid,
        in_specs=[row_spec, row_spec, mat_spec, vec_spec,
                  vec_spec, vec_spec, mat_spec, vec_spec],
        out_specs=row_spec,
        out_shape=jax.ShapeDtypeStruct((N_NODES, D), jnp.float32),
    )(agg_a, agg_b, w1, b1, gs, be, w2, b2)


def _mlp3_body(aa_ref, ab_ref, w1_ref, b1_ref, gs_ref, be_ref,
               w2_ref, b2_ref, batch_ref, gbn_ref, bbn_ref, wf_ref, bf_ref,
               out_ref, pool_acc):
    i = pl.program_id(0)
    h = aa_ref[...] + ab_ref[...]
    t = jnp.dot(h, w1_ref[...], preferred_element_type=jnp.float32)
    t = t + b1_ref[...]
    t = jnp.where(t >= 0, t, 0.2 * t)
    t = t * gs_ref[...] + be_ref[...]
    t = jnp.dot(t, w2_ref[...], preferred_element_type=jnp.float32)
    t = t + b2_ref[...]
    t = jnp.where(t >= 0, t, 0.2 * t)
    seg = batch_ref[0, 0, :]
    onehot = (seg[:, None] ==
              lax.broadcasted_iota(jnp.int32, (_MLP_R, N_GRAPHS), 1)
              ).astype(jnp.float32)
    part = lax.dot_general(onehot, t, (((0,), (0,)), ((), ())),
                           preferred_element_type=jnp.float32)

    @pl.when(i == 0)
    def _():
        pool_acc[...] = jnp.zeros_like(pool_acc)

    pool_acc[...] += part

    @pl.when(i == pl.num_programs(0) - 1)
    def _():
        p = pool_acc[...] * gbn_ref[...] + bbn_ref[...]
        out_ref[...] = jnp.dot(p, wf_ref[...],
                               preferred_element_type=jnp.float32) + bf_ref[...]


def _tc_mlp3_pool(agg_a, agg_b, w1, b1, gs, be, w2, b2,
                  batch3, gbn, bbn, wf, bf):
    grid = (N_NODES // _MLP_R,)
    row_spec = pl.BlockSpec((_MLP_R, D), lambda i: (i, 0))
    mat_spec = pl.BlockSpec((D, D), lambda i: (0, 0))
    vec_spec = pl.BlockSpec((1, D), lambda i: (0, 0))
    return pl.pallas_call(
        _mlp3_body,
        grid=grid,
        in_specs=[row_spec, row_spec, mat_spec, vec_spec,
                  vec_spec, vec_spec, mat_spec, vec_spec,
                  pl.BlockSpec((1, 1, _MLP_R), lambda i: (i, 0, 0)),
                  vec_spec, vec_spec,
                  pl.BlockSpec((D, LAT), lambda i: (0, 0)),
                  pl.BlockSpec((1, LAT), lambda i: (0, 0))],
        out_specs=pl.BlockSpec((N_GRAPHS, LAT), lambda i: (0, 0)),
        out_shape=jax.ShapeDtypeStruct((N_GRAPHS, LAT), jnp.float32),
        scratch_shapes=[pltpu.VMEM((N_GRAPHS, D), jnp.float32)],
    )(agg_a, agg_b, w1, b1, gs, be, w2, b2, batch3, gbn, bbn, wf, bf)


def kernel(x, edge_index, batch,
           W1_0, b1_0, g_0, be_0, W2_0, b2_0,
           W1_1, b1_1, g_1, be_1, W2_1, b2_1,
           W1_2, b1_2, g_2, be_2, W2_2, b2_2,
           g_bn, b_bn, Wf, bf):
    bn_scale = 1.0 / jnp.sqrt(jnp.float32(1.0 + 1e-5))
    pad = EWP - EW
    src = jnp.pad(edge_index[0].astype(jnp.int32).reshape(NW, EW),
                  ((0, 0), (0, pad))).reshape(NC, NS, ECH, EK)
    dst = jnp.pad(edge_index[1].astype(jnp.int32).reshape(NW, EW),
                  ((0, 0), (0, pad)),
                  constant_values=N_NODES).reshape(NC, NS, ECH, EK)
    batch3 = batch.astype(jnp.int32).reshape(N_NODES // _MLP_R, 1, _MLP_R)
    zeros = jnp.zeros((N_NODES, D), jnp.float32)

    def row(v):
        return v.reshape(1, -1).astype(jnp.float32)

    params = [
        (W1_0, row(b1_0), row(g_0) * bn_scale, row(be_0), W2_0, row(b2_0)),
        (W1_1, row(b1_1), row(g_1) * bn_scale, row(be_1), W2_1, row(b2_1)),
        (W1_2, row(b1_2), row(g_2) * bn_scale, row(be_2), W2_2, row(b2_2)),
    ]

    h = x
    for (w1, b1, gs, be, w2, b2) in params[:2]:
        agg = _sc_agg(h, src, dst, zeros)
        h = _tc_mlp(agg[0], agg[1], w1, b1, gs, be, w2, b2)

    (w1, b1, gs, be, w2, b2) = params[2]
    agg = _sc_agg(h, src, dst, zeros)
    return _tc_mlp3_pool(agg[0], agg[1], w1, b1, gs, be, w2, b2,
                         batch3, row(g_bn) * bn_scale, row(b_bn),
                         Wf, row(bf))


# speedup vs baseline: 1.0032x; 1.0032x over previous
import jax
import jax.numpy as jnp
from jax import lax
from jax.experimental import pallas as pl
from jax.experimental.pallas import tpu as pltpu
from jax.experimental.pallas import tpu_sc as plsc

N_NODES = 10000
N_EDGES = 320000
N_GRAPHS = 64
D = 128
LAT = 64

NC = 2
NS = 16
NW = NC * NS

EW = N_EDGES // NW
EK = 80
ECH = 128
EWP = ECH * EK
IBCH = 8
NBLK = ECH // IBCH
NB = 3
ADUMP = 8

PK = 104
PCH = 3
PW = PK * PCH
PTAIL = N_NODES - PW * NW

_SC_MESH = plsc.VectorSubcoreMesh(core_axis_name="c", subcore_axis_name="s")


ZR = 624
ZTAIL = N_NODES - NS * ZR


def _sc_agg_body(h_hbm, src_hbm, dst_hbm, zeros_hbm, out_hbm,
                 src_v, dst_v, rows_v, acc,
                 sem_ga, sem_gb, sem_gc, sem_sa, sem_sb, sem_sc,
                 sem_ia, sem_ib):
    c = lax.axis_index("c")
    s = lax.axis_index("s")

    @pl.when(c == 0)
    def _():
        pltpu.sync_copy(h_hbm.at[pl.ds(s * ZR, ZR)],
                        acc.at[pl.ds(s * ZR, ZR)])

        @pl.when(s == 0)
        def _():
            pltpu.sync_copy(h_hbm.at[pl.ds(NS * ZR, ZTAIL)],
                            acc.at[pl.ds(NS * ZR, ZTAIL)])

    @pl.when(c == 1)
    def _():
        pltpu.sync_copy(zeros_hbm.at[pl.ds(s * ZR, ZR)],
                        acc.at[pl.ds(s * ZR, ZR)])

        @pl.when(s == 0)
        def _():
            pltpu.sync_copy(zeros_hbm.at[pl.ds(NS * ZR, ZTAIL)],
                            acc.at[pl.ds(NS * ZR, ZTAIL)])

    rows = tuple(rows_v.at[i] for i in range(NB))
    gsem = (sem_ga, sem_gb, sem_gc)
    ssem = (sem_sa, sem_sb, sem_sc)
    srcb = (src_v.at[0], src_v.at[1])
    dstb = (dst_v.at[0], dst_v.at[1])

    def _idx_load(blk, sync=False):
        p = blk % 2
        sl = pl.ds(blk * IBCH, IBCH)
        if sync:
            pltpu.sync_copy(src_hbm.at[c, s, sl], srcb[p])
            pltpu.sync_copy(dst_hbm.at[c, s, sl], dstb[p])
        else:
            pltpu.async_copy(src_hbm.at[c, s, sl], srcb[p], sem_ia)
            pltpu.async_copy(dst_hbm.at[c, s, sl], dstb[p], sem_ib)

    def _idx_wait():
        sl = pl.ds(0, IBCH)
        pltpu.make_async_copy(src_hbm.at[c, s, sl], srcb[0], sem_ia).wait()
        pltpu.make_async_copy(dst_hbm.at[c, s, sl], dstb[0], sem_ib).wait()

    def _gather(k):
        p, j, b = (k // IBCH) % 2, k % IBCH, k % NB
        pltpu.async_copy(h_hbm.at[srcb[p].at[j]], rows[b], gsem[b])

    def _scatter(k):
        p, j, b = (k // IBCH) % 2, k % IBCH, k % NB
        pltpu.async_copy(rows[b], acc.at[dstb[p].at[j]], ssem[b], add=True)

    def _wait_g(k):
        b = k % NB
        pltpu.make_async_copy(h_hbm.at[srcb[0].at[0]], rows[b], gsem[b]).wait()

    def _wait_s(k):
        b = k % NB
        pltpu.make_async_copy(rows[b], acc.at[dstb[0].at[0]], ssem[b]).wait()

    _idx_load(0, sync=True)
    _idx_load(1)
    plsc.subcore_barrier()

    _gather(0)
    _gather(1)
    for k in range(ECH):
        _wait_g(k)
        _scatter(k)
        if k + 2 < ECH:
            if k >= 1:
                _wait_s(k - 1)
            if k % IBCH == 0 and k >= IBCH:
                nblk = k // IBCH + 1
                if nblk < NBLK:
                    _idx_load(nblk)
            if (k + 2) % IBCH == 0:
                _idx_wait()
            _gather(k + 2)
    _wait_s(ECH - 2)
    _wait_s(ECH - 1)
    plsc.subcore_barrier()
    pltpu.sync_copy(acc.at[pl.ds(s * ZR, ZR)],
                    out_hbm.at[c, pl.ds(s * ZR, ZR)])

    @pl.when(s == 0)
    def _():
        pltpu.sync_copy(acc.at[pl.ds(NS * ZR, ZTAIL)],
                        out_hbm.at[c, pl.ds(NS * ZR, ZTAIL)])


_sc_agg = pl.kernel(
    _sc_agg_body,
    out_type=jax.ShapeDtypeStruct((NC, N_NODES, D), jnp.float32),
    mesh=_SC_MESH,
    scratch_types=[
        pltpu.VMEM((2, IBCH, EK), jnp.int32),
        pltpu.VMEM((2, IBCH, EK), jnp.int32),
        pltpu.VMEM((NB, EK, D), jnp.float32),
        pltpu.VMEM_SHARED((N_NODES + ADUMP, D), jnp.float32),
        pltpu.SemaphoreType.DMA,
        pltpu.SemaphoreType.DMA,
        pltpu.SemaphoreType.DMA,
        pltpu.SemaphoreType.DMA,
        pltpu.SemaphoreType.DMA,
        pltpu.SemaphoreType.DMA,
        pltpu.SemaphoreType.DMA,
        pltpu.SemaphoreType.DMA,
    ],
)


_MLP_R = 1000


def _mlp_body(aa_ref, ab_ref, w1_ref, b1_ref, gs_ref, be_ref,
              w2_ref, b2_ref, out_ref):
    h = aa_ref[...] + ab_ref[...]
    t = jnp.dot(h, w1_ref[...], preferred_element_type=jnp.float32)
    t = t + b1_ref[...]
    t = jnp.where(t >= 0, t, 0.2 * t)
    t = t * gs_ref[...] + be_ref[...]
    t = jnp.dot(t, w2_ref[...], preferred_element_type=jnp.float32)
    t = t + b2_ref[...]
    out_ref[...] = jnp.where(t >= 0, t, 0.2 * t)


def _tc_mlp(agg_a, agg_b, w1, b1, gs, be, w2, b2):
    grid = (N_NODES // _MLP_R,)
    row_spec = pl.BlockSpec((_MLP_R, D), lambda i: (i, 0))
    mat_spec = pl.BlockSpec((D, D), lambda i: (0, 0))
    vec_spec = pl.BlockSpec((1, D), lambda i: (0, 0))
    return pl.pallas_call(
        _mlp_body,
        grid=grid,
        in_specs=[row_spec, row_spec, mat_spec, vec_spec,
                  vec_spec, vec_spec, mat_spec, vec_spec],
        out_specs=row_spec,
        out_shape=jax.ShapeDtypeStruct((N_NODES, D), jnp.float32),
    )(agg_a, agg_b, w1, b1, gs, be, w2, b2)


def _mlp3_body(aa_ref, ab_ref, w1_ref, b1_ref, gs_ref, be_ref,
               w2_ref, b2_ref, batch_ref, gbn_ref, bbn_ref, wf_ref, bf_ref,
               out_ref, pool_acc):
    i = pl.program_id(0)
    h = aa_ref[...] + ab_ref[...]
    t = jnp.dot(h, w1_ref[...], preferred_element_type=jnp.float32)
    t = t + b1_ref[...]
    t = jnp.where(t >= 0, t, 0.2 * t)
    t = t * gs_ref[...] + be_ref[...]
    t = jnp.dot(t, w2_ref[...], preferred_element_type=jnp.float32)
    t = t + b2_ref[...]
    t = jnp.where(t >= 0, t, 0.2 * t)
    seg = batch_ref[0, 0, :]
    onehot = (seg[:, None] ==
              lax.broadcasted_iota(jnp.int32, (_MLP_R, N_GRAPHS), 1)
              ).astype(jnp.float32)
    part = lax.dot_general(onehot, t, (((0,), (0,)), ((), ())),
                           preferred_element_type=jnp.float32)

    @pl.when(i == 0)
    def _():
        pool_acc[...] = jnp.zeros_like(pool_acc)

    pool_acc[...] += part

    @pl.when(i == pl.num_programs(0) - 1)
    def _():
        p = pool_acc[...] * gbn_ref[...] + bbn_ref[...]
        out_ref[...] = jnp.dot(p, wf_ref[...],
                               preferred_element_type=jnp.float32) + bf_ref[...]


def _tc_mlp3_pool(agg_a, agg_b, w1, b1, gs, be, w2, b2,
                  batch3, gbn, bbn, wf, bf):
    grid = (N_NODES // _MLP_R,)
    row_spec = pl.BlockSpec((_MLP_R, D), lambda i: (i, 0))
    mat_spec = pl.BlockSpec((D, D), lambda i: (0, 0))
    vec_spec = pl.BlockSpec((1, D), lambda i: (0, 0))
    return pl.pallas_call(
        _mlp3_body,
        grid=grid,
        in_specs=[row_spec, row_spec, mat_spec, vec_spec,
                  vec_spec, vec_spec, mat_spec, vec_spec,
                  pl.BlockSpec((1, 1, _MLP_R), lambda i: (i, 0, 0)),
                  vec_spec, vec_spec,
                  pl.BlockSpec((D, LAT), lambda i: (0, 0)),
                  pl.BlockSpec((1, LAT), lambda i: (0, 0))],
        out_specs=pl.BlockSpec((N_GRAPHS, LAT), lambda i: (0, 0)),
        out_shape=jax.ShapeDtypeStruct((N_GRAPHS, LAT), jnp.float32),
        scratch_shapes=[pltpu.VMEM((N_GRAPHS, D), jnp.float32)],
    )(agg_a, agg_b, w1, b1, gs, be, w2, b2, batch3, gbn, bbn, wf, bf)


def kernel(x, edge_index, batch,
           W1_0, b1_0, g_0, be_0, W2_0, b2_0,
           W1_1, b1_1, g_1, be_1, W2_1, b2_1,
           W1_2, b1_2, g_2, be_2, W2_2, b2_2,
           g_bn, b_bn, Wf, bf):
    bn_scale = 1.0 / jnp.sqrt(jnp.float32(1.0 + 1e-5))
    pad = EWP - EW
    src = jnp.pad(edge_index[0].astype(jnp.int32).reshape(NW, EW),
                  ((0, 0), (0, pad))).reshape(NC, NS, ECH, EK)
    pad_dst = jnp.broadcast_to(N_NODES + jnp.arange(pad, dtype=jnp.int32)
                               % ADUMP, (NW, pad))
    dst = jnp.concatenate(
        [edge_index[1].astype(jnp.int32).reshape(NW, EW), pad_dst],
        axis=1).reshape(NC, NS, ECH, EK)
    batch3 = batch.astype(jnp.int32).reshape(N_NODES // _MLP_R, 1, _MLP_R)
    zeros = jnp.zeros((N_NODES, D), jnp.float32)

    def row(v):
        return v.reshape(1, -1).astype(jnp.float32)

    params = [
        (W1_0, row(b1_0), row(g_0) * bn_scale, row(be_0), W2_0, row(b2_0)),
        (W1_1, row(b1_1), row(g_1) * bn_scale, row(be_1), W2_1, row(b2_1)),
        (W1_2, row(b1_2), row(g_2) * bn_scale, row(be_2), W2_2, row(b2_2)),
    ]

    h = x
    for (w1, b1, gs, be, w2, b2) in params[:2]:
        agg = _sc_agg(h, src, dst, zeros)
        h = _tc_mlp(agg[0], agg[1], w1, b1, gs, be, w2, b2)

    (w1, b1, gs, be, w2, b2) = params[2]
    agg = _sc_agg(h, src, dst, zeros)
    return _tc_mlp3_pool(agg[0], agg[1], w1, b1, gs, be, w2, b2,
                         batch3, row(g_bn) * bn_scale, row(b_bn),
                         Wf, row(bf))


# speedup vs baseline: 2.6454x; 2.6369x over previous
import jax
import jax.numpy as jnp
from jax import lax
from jax.experimental import pallas as pl
from jax.experimental.pallas import tpu as pltpu
from jax.experimental.pallas import tpu_sc as plsc

N_NODES = 10000
N_EDGES = 320000
N_GRAPHS = 64
D = 128
LAT = 64

NC = 2
NS = 16
NW = NC * NS

EW = N_EDGES // NW
EK = 125
ECH = EW // EK
IBCH = 8
NBLK = ECH // IBCH

PK = 104
PCH = 3
PW = PK * PCH
PTAIL = N_NODES - PW * NW

_SC_MESH = plsc.VectorSubcoreMesh(core_axis_name="c", subcore_axis_name="s")


ZR = 624
ZTAIL = N_NODES - NS * ZR


def _sc_agg_body(h_hbm, src_hbm, dst_hbm, zeros_hbm, out_hbm,
                 src_v, dst_v, rows_v, acc,
                 sem_ga, sem_gb, sem_sa, sem_sb, sem_ia, sem_ib):
    c = lax.axis_index("c")
    s = lax.axis_index("s")

    @pl.when(c == 0)
    def _():
        pltpu.sync_copy(h_hbm.at[pl.ds(s * ZR, ZR)],
                        acc.at[pl.ds(s * ZR, ZR)])

        @pl.when(s == 0)
        def _():
            pltpu.sync_copy(h_hbm.at[pl.ds(NS * ZR, ZTAIL)],
                            acc.at[pl.ds(NS * ZR, ZTAIL)])

    @pl.when(c == 1)
    def _():
        pltpu.sync_copy(zeros_hbm.at[pl.ds(s * ZR, ZR)],
                        acc.at[pl.ds(s * ZR, ZR)])

        @pl.when(s == 0)
        def _():
            pltpu.sync_copy(zeros_hbm.at[pl.ds(NS * ZR, ZTAIL)],
                            acc.at[pl.ds(NS * ZR, ZTAIL)])

    rows = (rows_v.at[0], rows_v.at[1])
    gsem = (sem_ga, sem_gb)
    ssem = (sem_sa, sem_sb)
    srcb = (src_v.at[0], src_v.at[1])
    dstb = (dst_v.at[0], dst_v.at[1])

    def _idx_load(blk, sync=False):
        p = blk % 2
        sl = pl.ds(blk * IBCH, IBCH)
        if sync:
            pltpu.sync_copy(src_hbm.at[c, s, sl], srcb[p])
            pltpu.sync_copy(dst_hbm.at[c, s, sl], dstb[p])
        else:
            pltpu.async_copy(src_hbm.at[c, s, sl], srcb[p], sem_ia)
            pltpu.async_copy(dst_hbm.at[c, s, sl], dstb[p], sem_ib)

    def _idx_wait():
        sl = pl.ds(0, IBCH)
        pltpu.make_async_copy(src_hbm.at[c, s, sl], srcb[0], sem_ia).wait()
        pltpu.make_async_copy(dst_hbm.at[c, s, sl], dstb[0], sem_ib).wait()

    def _gather(k):
        p, j, b = (k // IBCH) % 2, k % IBCH, k % 2
        pltpu.async_copy(h_hbm.at[srcb[p].at[j]], rows[b], gsem[b])

    def _scatter(k):
        p, j, b = (k // IBCH) % 2, k % IBCH, k % 2
        pltpu.async_copy(rows[b], acc.at[dstb[p].at[j]], ssem[b], add=True)

    def _wait_g(k):
        b = k % 2
        pltpu.make_async_copy(h_hbm.at[srcb[0].at[0]], rows[b], gsem[b]).wait()

    def _wait_s(k):
        b = k % 2
        pltpu.make_async_copy(rows[b], acc.at[dstb[0].at[0]], ssem[b]).wait()

    _idx_load(0, sync=True)
    _idx_load(1)
    plsc.subcore_barrier()

    _gather(0)
    _wait_g(0)
    _scatter(0)
    _gather(1)
    for k in range(1, ECH - 1):
        _wait_g(k)
        _scatter(k)
        _wait_s(k + 1)
        if k % IBCH == 0 and k >= IBCH:
            nblk = k // IBCH + 1
            if nblk < NBLK:
                _idx_load(nblk)
        if (k + 1) % IBCH == 0:
            _idx_wait()
        _gather(k + 1)
    _wait_g(ECH - 1)
    _scatter(ECH - 1)
    _wait_s(ECH - 2)
    _wait_s(ECH - 1)
    plsc.subcore_barrier()
    pltpu.sync_copy(acc.at[pl.ds(s * ZR, ZR)],
                    out_hbm.at[c, pl.ds(s * ZR, ZR)])

    @pl.when(s == 0)
    def _():
        pltpu.sync_copy(acc.at[pl.ds(NS * ZR, ZTAIL)],
                        out_hbm.at[c, pl.ds(NS * ZR, ZTAIL)])


_sc_agg = pl.kernel(
    _sc_agg_body,
    out_type=jax.ShapeDtypeStruct((NC, N_NODES, D), jnp.float32),
    mesh=_SC_MESH,
    scratch_types=[
        pltpu.VMEM((2, IBCH, EK), jnp.int32),
        pltpu.VMEM((2, IBCH, EK), jnp.int32),
        pltpu.VMEM((2, EK, D), jnp.float32),
        pltpu.VMEM_SHARED((N_NODES, D), jnp.float32),
        pltpu.SemaphoreType.DMA,
        pltpu.SemaphoreType.DMA,
        pltpu.SemaphoreType.DMA,
        pltpu.SemaphoreType.DMA,
        pltpu.SemaphoreType.DMA,
        pltpu.SemaphoreType.DMA,
    ],
)


_MLP_R = 1000


def _mlp_body(aa_ref, ab_ref, w1_ref, b1_ref, gs_ref, be_ref,
              w2_ref, b2_ref, out_ref):
    h = aa_ref[...] + ab_ref[...]
    t = jnp.dot(h, w1_ref[...], preferred_element_type=jnp.float32)
    t = t + b1_ref[...]
    t = jnp.where(t >= 0, t, 0.2 * t)
    t = t * gs_ref[...] + be_ref[...]
    t = jnp.dot(t, w2_ref[...], preferred_element_type=jnp.float32)
    t = t + b2_ref[...]
    out_ref[...] = jnp.where(t >= 0, t, 0.2 * t)


def _tc_mlp(agg_a, agg_b, w1, b1, gs, be, w2, b2):
    grid = (N_NODES // _MLP_R,)
    row_spec = pl.BlockSpec((_MLP_R, D), lambda i: (i, 0))
    mat_spec = pl.BlockSpec((D, D), lambda i: (0, 0))
    vec_spec = pl.BlockSpec((1, D), lambda i: (0, 0))
    return pl.pallas_call(
        _mlp_body,
        grid=grid,
        in_specs=[row_spec, row_spec, mat_spec, vec_spec,
                  vec_spec, vec_spec, mat_spec, vec_spec],
        out_specs=row_spec,
        out_shape=jax.ShapeDtypeStruct((N_NODES, D), jnp.float32),
    )(agg_a, agg_b, w1, b1, gs, be, w2, b2)


def _mlp3_body(aa_ref, ab_ref, w1_ref, b1_ref, gs_ref, be_ref,
               w2_ref, b2_ref, batch_ref, gbn_ref, bbn_ref, wf_ref, bf_ref,
               out_ref, pool_acc):
    i = pl.program_id(0)
    h = aa_ref[...] + ab_ref[...]
    t = jnp.dot(h, w1_ref[...], preferred_element_type=jnp.float32)
    t = t + b1_ref[...]
    t = jnp.where(t >= 0, t, 0.2 * t)
    t = t * gs_ref[...] + be_ref[...]
    t = jnp.dot(t, w2_ref[...], preferred_element_type=jnp.float32)
    t = t + b2_ref[...]
    t = jnp.where(t >= 0, t, 0.2 * t)
    seg = batch_ref[0, 0, :]
    onehot = (seg[:, None] ==
              lax.broadcasted_iota(jnp.int32, (_MLP_R, N_GRAPHS), 1)
              ).astype(jnp.float32)
    part = lax.dot_general(onehot, t, (((0,), (0,)), ((), ())),
                           preferred_element_type=jnp.float32)

    @pl.when(i == 0)
    def _():
        pool_acc[...] = jnp.zeros_like(pool_acc)

    pool_acc[...] += part

    @pl.when(i == pl.num_programs(0) - 1)
    def _():
        p = pool_acc[...] * gbn_ref[...] + bbn_ref[...]
        out_ref[...] = jnp.dot(p, wf_ref[...],
                               preferred_element_type=jnp.float32) + bf_ref[...]


def _tc_mlp3_pool(agg_a, agg_b, w1, b1, gs, be, w2, b2,
                  batch3, gbn, bbn, wf, bf):
    grid = (N_NODES // _MLP_R,)
    row_spec = pl.BlockSpec((_MLP_R, D), lambda i: (i, 0))
    mat_spec = pl.BlockSpec((D, D), lambda i: (0, 0))
    vec_spec = pl.BlockSpec((1, D), lambda i: (0, 0))
    return pl.pallas_call(
        _mlp3_body,
        grid=grid,
        in_specs=[row_spec, row_spec, mat_spec, vec_spec,
                  vec_spec, vec_spec, mat_spec, vec_spec,
                  pl.BlockSpec((1, 1, _MLP_R), lambda i: (i, 0, 0)),
                  vec_spec, vec_spec,
                  pl.BlockSpec((D, LAT), lambda i: (0, 0)),
                  pl.BlockSpec((1, LAT), lambda i: (0, 0))],
        out_specs=pl.BlockSpec((N_GRAPHS, LAT), lambda i: (0, 0)),
        out_shape=jax.ShapeDtypeStruct((N_GRAPHS, LAT), jnp.float32),
        scratch_shapes=[pltpu.VMEM((N_GRAPHS, D), jnp.float32)],
    )(agg_a, agg_b, w1, b1, gs, be, w2, b2, batch3, gbn, bbn, wf, bf)


def kernel(x, edge_index, batch,
           W1_0, b1_0, g_0, be_0, W2_0, b2_0,
           W1_1, b1_1, g_1, be_1, W2_1, b2_1,
           W1_2, b1_2, g_2, be_2, W2_2, b2_2,
           g_bn, b_bn, Wf, bf):
    bn_scale = 1.0 / jnp.sqrt(jnp.float32(1.0 + 1e-5))
    src = edge_index[0].astype(jnp.int32).reshape(NC, NS, ECH, EK)
    dst = edge_index[1].astype(jnp.int32).reshape(NC, NS, ECH, EK)
    batch3 = batch.astype(jnp.int32).reshape(N_NODES // _MLP_R, 1, _MLP_R)
    zeros = jnp.zeros((N_NODES, D), jnp.float32)

    def row(v):
        return v.reshape(1, -1).astype(jnp.float32)

    params = [
        (W1_0, row(b1_0), row(g_0) * bn_scale, row(be_0), W2_0, row(b2_0)),
        (W1_1, row(b1_1), row(g_1) * bn_scale, row(be_1), W2_1, row(b2_1)),
        (W1_2, row(b1_2), row(g_2) * bn_scale, row(be_2), W2_2, row(b2_2)),
    ]

    h = x
    for (w1, b1, gs, be, w2, b2) in params[:2]:
        agg = _sc_agg(h, src, dst, zeros)
        h = _tc_mlp(agg[0], agg[1], w1, b1, gs, be, w2, b2)

    (w1, b1, gs, be, w2, b2) = params[2]
    agg = _sc_agg(h, src, dst, zeros)
    return _tc_mlp3_pool(agg[0], agg[1], w1, b1, gs, be, w2, b2,
                         batch3, row(g_bn) * bn_scale, row(b_bn),
                         Wf, row(bf))
